# baseline (speedup 1.0000x reference)
import functools

import jax
import jax.numpy as jnp
from jax import lax
from jax.experimental import pallas as pl
from jax.experimental.pallas import tpu as pltpu
from jax.experimental.pallas import tpu_sc as plsc

EMB = 128
FOUT = 64
N_MM = 4096
N_DD = 2048
E_MM = 131072
E_DD = 65536

NC = 2
NS = 16
NW = NC * NS

F32 = jnp.float32
I32 = jnp.int32

_mesh = plsc.VectorSubcoreMesh(
    core_axis_name="c", subcore_axis_name="s", num_cores=NC, num_subcores=NS)

_GRAPHS = (
    dict(n=N_MM, e=E_MM),
    dict(n=N_MM, e=E_MM),
    dict(n=N_DD, e=E_DD),
    dict(n=N_DD, e=E_DD),
)
_ROWS_MM = E_MM // 128 // NW
_ROWS_DD = E_DD // 128 // NW

_ZERO16F = functools.partial(jnp.zeros, (16,), F32)


def _wid():
    return lax.axis_index("c") * NS + lax.axis_index("s")


def _newton_rsqrt(x):
    i = lax.bitcast_convert_type(x, I32)
    i = jnp.int32(0x5F3759DF) - (i >> 1)
    y = lax.bitcast_convert_type(i, F32)
    for _ in range(3):
        y = y * (1.5 - 0.5 * x * y * y)
    return y


def _prep_body(mat_mf, e_mf, mat_mg, e_mg, mat_ds, e_ds, mat_dg, e_dg,
               nrm_mf_o, di2_mf_o, nrm_mg_o, di2_mg_o,
               nrm_ds_o, di2_ds_o, nrm_dg_o, di2_dg_o,
               ix_v, dd_v, ew_v, zz_v, pa_v, dv_v, deg4k, deg2k, sem):
    cid = lax.axis_index("c")
    sid = lax.axis_index("s")

    def _zz(i, carry):
        zz_v[pl.ds(i * 16, 16)] = _ZERO16F()
        return carry
    lax.fori_loop(0, 16, _zz, 0)

    for degsh, n in ((deg4k, N_MM), (deg2k, N_DD)):
        chunk = n // NS
        pltpu.sync_copy(zz_v.at[pl.ds(0, chunk)],
                        degsh.at[pl.ds(sid * chunk, chunk)])
    plsc.subcore_barrier()

    def _graph(mat, e_hbm, nrm_o, di2_o, degsh, n, e):
        shift = n.bit_length() - 1
        nrows = e // 128 // NS
        rbase = sid * nrows
        pltpu.sync_copy(e_hbm.at[0, pl.ds(rbase, nrows)],
                        ix_v.at[pl.ds(0, nrows)])
        pltpu.sync_copy(e_hbm.at[1, pl.ds(rbase, nrows)],
                        dd_v.at[pl.ds(0, nrows)])

        def _flat(r):
            for k in range(8):
                sl = pl.ds(k * 16, 16)
                ix_v[r, sl] = ix_v[r, sl] * n + dd_v[r, sl]
        plsc.parallel_loop(0, nrows, 1, unroll=2)(_flat)

        def _fire(j, carry):
            pltpu.async_copy(mat.at[ix_v.at[j]], ew_v.at[j], sem)
            return carry
        lax.fori_loop(0, nrows, _fire, 0)

        def _drain(j, carry):
            pltpu.make_async_copy(mat.at[ix_v.at[j]], ew_v.at[j], sem).wait()
            return carry
        lax.fori_loop(0, nrows, _drain, 0)

        def _scat(j, carry):
            pltpu.sync_copy(ew_v.at[j], degsh.at[dd_v.at[j]], add=True)
            return carry
        lax.fori_loop(0, nrows, _scat, 0)
        plsc.subcore_barrier()

        pltpu.sync_copy(degsh, pa_v.at[pl.ds(0, n)])

        def _dv(i):
            sl = pl.ds(i * 16, 16)
            x = pa_v[sl] + 1.0
            y = _newton_rsqrt(x)
            dv_v[sl] = y
            pa_v[sl] = y * y
        plsc.parallel_loop(0, n // 16, 1, unroll=4)(_dv)

        @pl.when(sid == 0)
        def _():
            pltpu.sync_copy(pa_v.at[pl.ds(0, n)], di2_o)

        def _nrow(r):
            for k in range(8):
                sl = pl.ds(k * 16, 16)
                s16 = lax.shift_right_logical(ix_v[r, sl], shift)
                dsv = plsc.load_gather(dv_v, [s16])
                ddv = plsc.load_gather(dv_v, [dd_v[r, sl]])
                ew_v[r, sl] = dsv * ew_v[r, sl] * ddv
        plsc.parallel_loop(0, nrows, 1, unroll=2)(_nrow)

        pltpu.sync_copy(ew_v.at[pl.ds(0, nrows)], nrm_o.at[pl.ds(rbase, nrows)])

    @pl.when(cid == 0)
    def _c0():
        _graph(mat_mf, e_mf, nrm_mf_o, di2_mf_o, deg4k, N_MM, E_MM)
        _graph(mat_ds, e_ds, nrm_ds_o, di2_ds_o, deg2k, N_DD, E_DD)

    @pl.when(cid == 1)
    def _c1():
        _graph(mat_mg, e_mg, nrm_mg_o, di2_mg_o, deg4k, N_MM, E_MM)
        _graph(mat_dg, e_dg, nrm_dg_o, di2_dg_o, deg2k, N_DD, E_DD)


_prep = pl.kernel(
    _prep_body,
    out_type=[
        jax.ShapeDtypeStruct((E_MM // 128, 128), F32),
        jax.ShapeDtypeStruct((N_MM,), F32),
        jax.ShapeDtypeStruct((E_MM // 128, 128), F32),
        jax.ShapeDtypeStruct((N_MM,), F32),
        jax.ShapeDtypeStruct((E_DD // 128, 128), F32),
        jax.ShapeDtypeStruct((N_DD,), F32),
        jax.ShapeDtypeStruct((E_DD // 128, 128), F32),
        jax.ShapeDtypeStruct((N_DD,), F32),
    ],
    mesh=_mesh,
    compiler_params=pltpu.CompilerParams(needs_layout_passes=False),
    scratch_types=[
        pltpu.VMEM((E_MM // 128 // NS, 128), I32),
        pltpu.VMEM((E_MM // 128 // NS, 128), I32),
        pltpu.VMEM((E_MM // 128 // NS, 128), F32),
        pltpu.VMEM((256,), F32),
        pltpu.VMEM((N_MM,), F32),
        pltpu.VMEM((N_MM,), F32),
        pltpu.VMEM_SHARED((N_MM,), F32),
        pltpu.VMEM_SHARED((N_DD,), F32),
        pltpu.SemaphoreType.DMA,
    ],
)


def _msg_body(xw_mf, e_mf, n_mf, xw_mg, e_mg, n_mg,
              xw_ds, e_ds, n_ds, xw_dg, e_dg, n_dg,
              o_mf, o_mg, o_ds, o_dg,
              si_v, di_v, nw_v, rows_a, rows_b, sc_a, sc_b, z_v,
              acc4k,
              gsem_a, gsem_b, ssem_a, ssem_b):
    cid = lax.axis_index("c")
    sid = lax.axis_index("s")

    def _zrow(r, carry):
        for k in range(8):
            z_v[r, pl.ds(k * 16, 16)] = _ZERO16F()
        return carry
    lax.fori_loop(0, 32, _zrow, 0)

    def _zero_own(acc, n):
        rpt = n // NS

        def _zb(b, carry):
            pltpu.sync_copy(z_v, acc.at[pl.ds(sid * rpt + b * 32, 32)])
            return carry
        lax.fori_loop(0, rpt // 32, _zb, 0)

    _zero_own(acc4k, N_MM)
    plsc.subcore_barrier()

    bufs = ((rows_a, sc_a, gsem_a, ssem_a), (rows_b, sc_b, gsem_b, ssem_b))

    def _graph(xw_hbm, e_hbm, n_hbm, acc, e):
        nrows = e // 128 // NS
        rbase = sid * nrows
        pltpu.sync_copy(e_hbm.at[0, pl.ds(rbase, nrows)],
                        si_v.at[pl.ds(0, nrows)])
        pltpu.sync_copy(e_hbm.at[1, pl.ds(rbase, nrows)],
                        di_v.at[pl.ds(0, nrows)])
        pltpu.sync_copy(n_hbm.at[pl.ds(rbase, nrows)],
                        nw_v.at[pl.ds(0, nrows)])

        pltpu.async_copy(xw_hbm.at[si_v.at[0]], rows_a, gsem_a)
        pltpu.async_copy(xw_hbm.at[si_v.at[1]], rows_b, gsem_b)

        def _pair(j2, carry):
            for b, (rows, scb, gsem, ssem) in enumerate(bufs):
                j = j2 * 2 + b
                pltpu.make_async_copy(xw_hbm.at[si_v.at[j]], rows, gsem).wait()

                @pl.when(j2 > 0)
                def _ws(scb=scb, ssem=ssem, j=j):
                    pltpu.make_async_copy(
                        scb, acc.at[di_v.at[j - 2]], ssem).wait()

                def _rs(r, rows=rows, scb=scb, j=j):
                    wv = plsc.load_gather(
                        nw_v, [jnp.full((16,), j, I32),
                               jnp.full((16,), r, I32)])
                    for k in range(8):
                        sl = pl.ds(k * 16, 16)
                        scb[r, sl] = rows[r, sl] * wv
                plsc.parallel_loop(0, 128, 1, unroll=32)(_rs)

                @pl.when(j + 2 < nrows)
                def _g2(j=j, rows=rows, gsem=gsem):
                    pltpu.async_copy(xw_hbm.at[si_v.at[j + 2]], rows, gsem)

                pltpu.async_copy(scb, acc.at[di_v.at[j]], ssem, add=True)
            return carry
        lax.fori_loop(0, nrows // 2, _pair, 0)

        pltpu.make_async_copy(sc_a, acc.at[di_v.at[nrows - 2]], ssem_a).wait()
        pltpu.make_async_copy(sc_b, acc.at[di_v.at[nrows - 1]], ssem_b).wait()

    def _writeout_own(acc, out, n):
        rpt = n // NS
        pltpu.sync_copy(acc.at[pl.ds(sid * rpt, rpt)],
                        out.at[pl.ds(sid * rpt, rpt)])

    @pl.when(cid == 0)
    def _g0():
        _graph(xw_mf, e_mf, n_mf, acc4k, E_MM)

    @pl.when(cid == 1)
    def _g1():
        _graph(xw_mg, e_mg, n_mg, acc4k, E_MM)
    plsc.subcore_barrier()

    @pl.when(cid == 0)
    def _w0():
        _writeout_own(acc4k, o_mf, N_MM)

    @pl.when(cid == 1)
    def _w1():
        _writeout_own(acc4k, o_mg, N_MM)
    plsc.subcore_barrier()
    _zero_own(acc4k, N_DD)
    plsc.subcore_barrier()

    @pl.when(cid == 0)
    def _g0d():
        _graph(xw_ds, e_ds, n_ds, acc4k, E_DD)

    @pl.when(cid == 1)
    def _g1d():
        _graph(xw_dg, e_dg, n_dg, acc4k, E_DD)
    plsc.subcore_barrier()

    @pl.when(cid == 0)
    def _w0d():
        _writeout_own(acc4k, o_ds, N_DD)

    @pl.when(cid == 1)
    def _w1d():
        _writeout_own(acc4k, o_dg, N_DD)


_msg = pl.kernel(
    _msg_body,
    out_type=[
        jax.ShapeDtypeStruct((N_MM, EMB), F32),
        jax.ShapeDtypeStruct((N_MM, EMB), F32),
        jax.ShapeDtypeStruct((N_DD, EMB), F32),
        jax.ShapeDtypeStruct((N_DD, EMB), F32),
    ],
    mesh=_mesh,
    compiler_params=pltpu.CompilerParams(needs_layout_passes=False),
    scratch_types=[
        pltpu.VMEM((E_MM // 128 // NS, 128), I32),
        pltpu.VMEM((E_MM // 128 // NS, 128), I32),
        pltpu.VMEM((E_MM // 128 // NS, 128), F32),
        pltpu.VMEM((128, EMB), F32),
        pltpu.VMEM((128, EMB), F32),
        pltpu.VMEM((128, EMB), F32),
        pltpu.VMEM((128, EMB), F32),
        pltpu.VMEM((32, EMB), F32),
        pltpu.VMEM_SHARED((N_MM, EMB), F32),
        pltpu.SemaphoreType.DMA,
        pltpu.SemaphoreType.DMA,
        pltpu.SemaphoreType.DMA,
        pltpu.SemaphoreType.DMA,
    ],
)


def _tc1_body(a_ref, b_ref, wmf, wmg, wds, wdg, omf, omg, ods, odg):
    a = a_ref[...]
    b = b_ref[...]
    omf[...] = jnp.dot(a, wmf[...], preferred_element_type=F32)
    omg[...] = jnp.dot(a, wmg[...], preferred_element_type=F32)
    ods[...] = jnp.dot(b, wds[...], preferred_element_type=F32)
    odg[...] = jnp.dot(b, wdg[...], preferred_element_type=F32)


_tc1 = pl.pallas_call(
    _tc1_body,
    out_shape=[
        jax.ShapeDtypeStruct((N_MM, EMB), F32),
        jax.ShapeDtypeStruct((N_MM, EMB), F32),
        jax.ShapeDtypeStruct((N_DD, EMB), F32),
        jax.ShapeDtypeStruct((N_DD, EMB), F32),
    ],
)


def _tc2_body(*refs):
    for g in range(4):
        s_ref, xw_ref, di2_ref, b_ref, w_ref = refs[5 * g:5 * g + 5]
        o_ref = refs[20 + g]
        h = s_ref[...] + di2_ref[...] * xw_ref[...] + b_ref[...]
        h = jnp.maximum(h, 0.0)
        o_ref[...] = jnp.dot(h, w_ref[...], preferred_element_type=F32)


_tc2 = pl.pallas_call(
    _tc2_body,
    out_shape=[
        jax.ShapeDtypeStruct((N_MM, EMB), F32),
        jax.ShapeDtypeStruct((N_MM, EMB), F32),
        jax.ShapeDtypeStruct((N_DD, EMB), F32),
        jax.ShapeDtypeStruct((N_DD, EMB), F32),
    ],
)


def _tc3_body(*refs):
    hs = []
    for g in range(4):
        s_ref, xw_ref, di2_ref, b_ref = refs[4 * g:4 * g + 4]
        h = s_ref[...] + di2_ref[...] * xw_ref[...] + b_ref[...]
        hs.append(jnp.maximum(h, 0.0))
    wlmf, blmf, wlmg, blmg = refs[16:20]
    mi_ref, dis_ref = refs[20], refs[21]
    mmf = jnp.dot(hs[0], wlmf[...], preferred_element_type=F32) + blmf[...]
    mmg = jnp.dot(hs[1], wlmg[...], preferred_element_type=F32) + blmg[...]
    dds = jnp.dot(hs[2], wlmf[...], preferred_element_type=F32) + blmf[...]
    ddg = jnp.dot(hs[3], wlmg[...], preferred_element_type=F32) + blmg[...]
    mw = (mmf > 0).astype(F32)
    dw = (dds > 0).astype(F32)
    mi_ref[...] = mw * mmf + (1.0 - mw) * mmg
    dis_ref[...] = dw * dds + (1.0 - dw) * ddg


_tc3 = pl.pallas_call(
    _tc3_body,
    out_shape=[
        jax.ShapeDtypeStruct((N_MM, FOUT), F32),
        jax.ShapeDtypeStruct((N_DD, FOUT), F32),
    ],
)


def _tc4_body(mi_ref, dis_ref, o_ref):
    o_ref[...] = lax.dot_general(
        mi_ref[...], dis_ref[...],
        dimension_numbers=(((1,), (1,)), ((), ())),
        preferred_element_type=F32)


_tc4 = pl.pallas_call(
    _tc4_body,
    out_shape=jax.ShapeDtypeStruct((N_MM, N_DD), F32),
)


def kernel(mm0, dd0, mm_func_edges, mm_func_matrix, mm_gip_edges,
           mm_gip_matrix, dd_sema_edges, dd_sema_matrix, dd_gip_edges,
           dd_gip_matrix, W1mf, b1mf, W2mf, b2mf, W1mg, b1mg, W2mg, b2mg,
           W1ds, b1ds, W2ds, b2ds, W1dg, b1dg, W2dg, b2dg,
           Wlmf, blmf, Wlmg, blmg):
    e_mf = mm_func_edges.astype(I32).reshape(2, -1, 128)
    e_mg = mm_gip_edges.astype(I32).reshape(2, -1, 128)
    e_ds = dd_sema_edges.astype(I32).reshape(2, -1, 128)
    e_dg = dd_gip_edges.astype(I32).reshape(2, -1, 128)

    (nrm_mf, di2_mf, nrm_mg, di2_mg, nrm_ds, di2_ds, nrm_dg, di2_dg) = _prep(
        mm_func_matrix.reshape(-1), e_mf,
        mm_gip_matrix.reshape(-1), e_mg,
        dd_sema_matrix.reshape(-1), e_ds,
        dd_gip_matrix.reshape(-1), e_dg)

    xw1_mf, xw1_mg, xw1_ds, xw1_dg = _tc1(mm0, dd0, W1mf, W1mg, W1ds, W1dg)

    sp1_mf, sp1_mg, sp1_ds, sp1_dg = _msg(
        xw1_mf, e_mf, nrm_mf, xw1_mg, e_mg, nrm_mg,
        xw1_ds, e_ds, nrm_ds, xw1_dg, e_dg, nrm_dg)

    di2c_mf = di2_mf.reshape(-1, 1)
    di2c_mg = di2_mg.reshape(-1, 1)
    di2c_ds = di2_ds.reshape(-1, 1)
    di2c_dg = di2_dg.reshape(-1, 1)

    xw2_mf, xw2_mg, xw2_ds, xw2_dg = _tc2(
        sp1_mf, xw1_mf, di2c_mf, b1mf.reshape(1, -1), W2mf,
        sp1_mg, xw1_mg, di2c_mg, b1mg.reshape(1, -1), W2mg,
        sp1_ds, xw1_ds, di2c_ds, b1ds.reshape(1, -1), W2ds,
        sp1_dg, xw1_dg, di2c_dg, b1dg.reshape(1, -1), W2dg)

    sp2_mf, sp2_mg, sp2_ds, sp2_dg = _msg(
        xw2_mf, e_mf, nrm_mf, xw2_mg, e_mg, nrm_mg,
        xw2_ds, e_ds, nrm_ds, xw2_dg, e_dg, nrm_dg)

    mi, dis = _tc3(
        sp2_mf, xw2_mf, di2c_mf, b2mf.reshape(1, -1),
        sp2_mg, xw2_mg, di2c_mg, b2mg.reshape(1, -1),
        sp2_ds, xw2_ds, di2c_ds, b2ds.reshape(1, -1),
        sp2_dg, xw2_dg, di2c_dg, b2dg.reshape(1, -1),
        Wlmf, blmf.reshape(1, -1), Wlmg, blmg.reshape(1, -1))

    prod = _tc4(mi, dis)
    return (prod, mi, dis)


# speedup vs baseline: 1.0098x; 1.0098x over previous
import functools

import jax
import jax.numpy as jnp
from jax import lax
from jax.experimental import pallas as pl
from jax.experimental.pallas import tpu as pltpu
from jax.experimental.pallas import tpu_sc as plsc

EMB = 128
FOUT = 64
N_MM = 4096
N_DD = 2048
E_MM = 131072
E_DD = 65536

NC = 2
NS = 16
NW = NC * NS

F32 = jnp.float32
I32 = jnp.int32

_mesh = plsc.VectorSubcoreMesh(
    core_axis_name="c", subcore_axis_name="s", num_cores=NC, num_subcores=NS)

_GRAPHS = (
    dict(n=N_MM, e=E_MM),
    dict(n=N_MM, e=E_MM),
    dict(n=N_DD, e=E_DD),
    dict(n=N_DD, e=E_DD),
)
_ROWS_MM = E_MM // 128 // NW
_ROWS_DD = E_DD // 128 // NW

_ZERO16F = functools.partial(jnp.zeros, (16,), F32)


def _wid():
    return lax.axis_index("c") * NS + lax.axis_index("s")


def _newton_rsqrt(x):
    i = lax.bitcast_convert_type(x, I32)
    i = jnp.int32(0x5F3759DF) - (i >> 1)
    y = lax.bitcast_convert_type(i, F32)
    for _ in range(3):
        y = y * (1.5 - 0.5 * x * y * y)
    return y


def _prep_body(mat_mf, e_mf, mat_mg, e_mg, mat_ds, e_ds, mat_dg, e_dg,
               nrm_mf_o, di2_mf_o, nrm_mg_o, di2_mg_o,
               nrm_ds_o, di2_ds_o, nrm_dg_o, di2_dg_o,
               ix_v, dd_v, ew_v, zz_v, pa_v, dv_v, deg4k, deg2k, sem):
    cid = lax.axis_index("c")
    sid = lax.axis_index("s")

    def _zz(i, carry):
        zz_v[pl.ds(i * 16, 16)] = _ZERO16F()
        return carry
    lax.fori_loop(0, 16, _zz, 0)

    for degsh, n in ((deg4k, N_MM), (deg2k, N_DD)):
        chunk = n // NS
        pltpu.sync_copy(zz_v.at[pl.ds(0, chunk)],
                        degsh.at[pl.ds(sid * chunk, chunk)])
    plsc.subcore_barrier()

    def _graph(mat, e_hbm, nrm_o, di2_o, degsh, n, e):
        shift = n.bit_length() - 1
        nrows = e // 128 // NS
        rbase = sid * nrows
        pltpu.sync_copy(e_hbm.at[0, pl.ds(rbase, nrows)],
                        ix_v.at[pl.ds(0, nrows)])
        pltpu.sync_copy(e_hbm.at[1, pl.ds(rbase, nrows)],
                        dd_v.at[pl.ds(0, nrows)])

        def _flat(r):
            for k in range(8):
                sl = pl.ds(k * 16, 16)
                ix_v[r, sl] = ix_v[r, sl] * n + dd_v[r, sl]
        plsc.parallel_loop(0, nrows, 1, unroll=4)(_flat)

        def _fire(j, carry):
            pltpu.async_copy(mat.at[ix_v.at[j]], ew_v.at[j], sem)
            return carry
        lax.fori_loop(0, nrows, _fire, 0)

        def _drain(j, carry):
            pltpu.make_async_copy(mat.at[ix_v.at[j]], ew_v.at[j], sem).wait()
            return carry
        lax.fori_loop(0, nrows, _drain, 0)

        def _scat(j, carry):
            pltpu.sync_copy(ew_v.at[j], degsh.at[dd_v.at[j]], add=True)
            return carry
        lax.fori_loop(0, nrows, _scat, 0)
        plsc.subcore_barrier()

        pltpu.sync_copy(degsh, pa_v.at[pl.ds(0, n)])

        def _dv(i):
            sl = pl.ds(i * 16, 16)
            x = pa_v[sl] + 1.0
            y = _newton_rsqrt(x)
            dv_v[sl] = y
            pa_v[sl] = y * y
        plsc.parallel_loop(0, n // 16, 1, unroll=4)(_dv)

        @pl.when(sid == 0)
        def _():
            pltpu.sync_copy(pa_v.at[pl.ds(0, n)], di2_o)

        def _nrow(r):
            for k in range(8):
                sl = pl.ds(k * 16, 16)
                s16 = lax.shift_right_logical(ix_v[r, sl], shift)
                dsv = plsc.load_gather(dv_v, [s16])
                ddv = plsc.load_gather(dv_v, [dd_v[r, sl]])
                ew_v[r, sl] = dsv * ew_v[r, sl] * ddv
        plsc.parallel_loop(0, nrows, 1, unroll=4)(_nrow)

        pltpu.sync_copy(ew_v.at[pl.ds(0, nrows)], nrm_o.at[pl.ds(rbase, nrows)])

    @pl.when(cid == 0)
    def _c0():
        _graph(mat_mf, e_mf, nrm_mf_o, di2_mf_o, deg4k, N_MM, E_MM)
        _graph(mat_ds, e_ds, nrm_ds_o, di2_ds_o, deg2k, N_DD, E_DD)

    @pl.when(cid == 1)
    def _c1():
        _graph(mat_mg, e_mg, nrm_mg_o, di2_mg_o, deg4k, N_MM, E_MM)
        _graph(mat_dg, e_dg, nrm_dg_o, di2_dg_o, deg2k, N_DD, E_DD)


_prep = pl.kernel(
    _prep_body,
    out_type=[
        jax.ShapeDtypeStruct((E_MM // 128, 128), F32),
        jax.ShapeDtypeStruct((N_MM,), F32),
        jax.ShapeDtypeStruct((E_MM // 128, 128), F32),
        jax.ShapeDtypeStruct((N_MM,), F32),
        jax.ShapeDtypeStruct((E_DD // 128, 128), F32),
        jax.ShapeDtypeStruct((N_DD,), F32),
        jax.ShapeDtypeStruct((E_DD // 128, 128), F32),
        jax.ShapeDtypeStruct((N_DD,), F32),
    ],
    mesh=_mesh,
    compiler_params=pltpu.CompilerParams(needs_layout_passes=False),
    scratch_types=[
        pltpu.VMEM((E_MM // 128 // NS, 128), I32),
        pltpu.VMEM((E_MM // 128 // NS, 128), I32),
        pltpu.VMEM((E_MM // 128 // NS, 128), F32),
        pltpu.VMEM((256,), F32),
        pltpu.VMEM((N_MM,), F32),
        pltpu.VMEM((N_MM,), F32),
        pltpu.VMEM_SHARED((N_MM,), F32),
        pltpu.VMEM_SHARED((N_DD,), F32),
        pltpu.SemaphoreType.DMA,
    ],
)


def _msg_body(xw_mf, e_mf, n_mf, xw_mg, e_mg, n_mg,
              xw_ds, e_ds, n_ds, xw_dg, e_dg, n_dg,
              o_mf, o_mg, o_ds, o_dg,
              si_v, di_v, nw_v, rows_a, rows_b, sc_a, sc_b, z_v,
              acc4k,
              gsem_a, gsem_b, ssem_a, ssem_b):
    cid = lax.axis_index("c")
    sid = lax.axis_index("s")

    def _zrow(r, carry):
        for k in range(8):
            z_v[r, pl.ds(k * 16, 16)] = _ZERO16F()
        return carry
    lax.fori_loop(0, 32, _zrow, 0)

    def _zero_own(acc, n):
        rpt = n // NS

        def _zb(b, carry):
            pltpu.sync_copy(z_v, acc.at[pl.ds(sid * rpt + b * 32, 32)])
            return carry
        lax.fori_loop(0, rpt // 32, _zb, 0)

    _zero_own(acc4k, N_MM)
    plsc.subcore_barrier()

    bufs = ((rows_a, sc_a, gsem_a, ssem_a), (rows_b, sc_b, gsem_b, ssem_b))

    def _graph(xw_hbm, e_hbm, n_hbm, acc, e):
        nrows = e // 128 // NS
        rbase = sid * nrows
        pltpu.sync_copy(e_hbm.at[0, pl.ds(rbase, nrows)],
                        si_v.at[pl.ds(0, nrows)])
        pltpu.sync_copy(e_hbm.at[1, pl.ds(rbase, nrows)],
                        di_v.at[pl.ds(0, nrows)])
        pltpu.sync_copy(n_hbm.at[pl.ds(rbase, nrows)],
                        nw_v.at[pl.ds(0, nrows)])

        pltpu.async_copy(xw_hbm.at[si_v.at[0]], rows_a, gsem_a)
        pltpu.async_copy(xw_hbm.at[si_v.at[1]], rows_b, gsem_b)

        def _pair(j2, carry):
            for b, (rows, scb, gsem, ssem) in enumerate(bufs):
                j = j2 * 2 + b
                pltpu.make_async_copy(xw_hbm.at[si_v.at[j]], rows, gsem).wait()

                @pl.when(j2 > 0)
                def _ws(scb=scb, ssem=ssem, j=j):
                    pltpu.make_async_copy(
                        scb, acc.at[di_v.at[j - 2]], ssem).wait()

                def _rs(r, rows=rows, scb=scb, j=j):
                    wv = plsc.load_gather(
                        nw_v, [jnp.full((16,), j, I32),
                               jnp.full((16,), r, I32)])
                    for k in range(8):
                        sl = pl.ds(k * 16, 16)
                        scb[r, sl] = rows[r, sl] * wv
                plsc.parallel_loop(0, 128, 1, unroll=16)(_rs)

                @pl.when(j + 2 < nrows)
                def _g2(j=j, rows=rows, gsem=gsem):
                    pltpu.async_copy(xw_hbm.at[si_v.at[j + 2]], rows, gsem)

                pltpu.async_copy(scb, acc.at[di_v.at[j]], ssem, add=True)
            return carry
        lax.fori_loop(0, nrows // 2, _pair, 0)

        pltpu.make_async_copy(sc_a, acc.at[di_v.at[nrows - 2]], ssem_a).wait()
        pltpu.make_async_copy(sc_b, acc.at[di_v.at[nrows - 1]], ssem_b).wait()

    def _writeout_own(acc, out, n):
        rpt = n // NS
        pltpu.sync_copy(acc.at[pl.ds(sid * rpt, rpt)],
                        out.at[pl.ds(sid * rpt, rpt)])

    @pl.when(cid == 0)
    def _g0():
        _graph(xw_mf, e_mf, n_mf, acc4k, E_MM)

    @pl.when(cid == 1)
    def _g1():
        _graph(xw_mg, e_mg, n_mg, acc4k, E_MM)
    plsc.subcore_barrier()

    @pl.when(cid == 0)
    def _w0():
        _writeout_own(acc4k, o_mf, N_MM)

    @pl.when(cid == 1)
    def _w1():
        _writeout_own(acc4k, o_mg, N_MM)
    plsc.subcore_barrier()
    _zero_own(acc4k, N_DD)
    plsc.subcore_barrier()

    @pl.when(cid == 0)
    def _g0d():
        _graph(xw_ds, e_ds, n_ds, acc4k, E_DD)

    @pl.when(cid == 1)
    def _g1d():
        _graph(xw_dg, e_dg, n_dg, acc4k, E_DD)
    plsc.subcore_barrier()

    @pl.when(cid == 0)
    def _w0d():
        _writeout_own(acc4k, o_ds, N_DD)

    @pl.when(cid == 1)
    def _w1d():
        _writeout_own(acc4k, o_dg, N_DD)


_msg = pl.kernel(
    _msg_body,
    out_type=[
        jax.ShapeDtypeStruct((N_MM, EMB), F32),
        jax.ShapeDtypeStruct((N_MM, EMB), F32),
        jax.ShapeDtypeStruct((N_DD, EMB), F32),
        jax.ShapeDtypeStruct((N_DD, EMB), F32),
    ],
    mesh=_mesh,
    compiler_params=pltpu.CompilerParams(needs_layout_passes=False),
    scratch_types=[
        pltpu.VMEM((E_MM // 128 // NS, 128), I32),
        pltpu.VMEM((E_MM // 128 // NS, 128), I32),
        pltpu.VMEM((E_MM // 128 // NS, 128), F32),
        pltpu.VMEM((128, EMB), F32),
        pltpu.VMEM((128, EMB), F32),
        pltpu.VMEM((128, EMB), F32),
        pltpu.VMEM((128, EMB), F32),
        pltpu.VMEM((32, EMB), F32),
        pltpu.VMEM_SHARED((N_MM, EMB), F32),
        pltpu.SemaphoreType.DMA,
        pltpu.SemaphoreType.DMA,
        pltpu.SemaphoreType.DMA,
        pltpu.SemaphoreType.DMA,
    ],
)


def _tc1_body(a_ref, b_ref, wmf, wmg, wds, wdg, omf, omg, ods, odg):
    a = a_ref[...]
    b = b_ref[...]
    omf[...] = jnp.dot(a, wmf[...], preferred_element_type=F32)
    omg[...] = jnp.dot(a, wmg[...], preferred_element_type=F32)
    ods[...] = jnp.dot(b, wds[...], preferred_element_type=F32)
    odg[...] = jnp.dot(b, wdg[...], preferred_element_type=F32)


_tc1 = pl.pallas_call(
    _tc1_body,
    out_shape=[
        jax.ShapeDtypeStruct((N_MM, EMB), F32),
        jax.ShapeDtypeStruct((N_MM, EMB), F32),
        jax.ShapeDtypeStruct((N_DD, EMB), F32),
        jax.ShapeDtypeStruct((N_DD, EMB), F32),
    ],
)


def _tc2_body(*refs):
    for g in range(4):
        s_ref, xw_ref, di2_ref, b_ref, w_ref = refs[5 * g:5 * g + 5]
        o_ref = refs[20 + g]
        h = s_ref[...] + di2_ref[...] * xw_ref[...] + b_ref[...]
        h = jnp.maximum(h, 0.0)
        o_ref[...] = jnp.dot(h, w_ref[...], preferred_element_type=F32)


_tc2 = pl.pallas_call(
    _tc2_body,
    out_shape=[
        jax.ShapeDtypeStruct((N_MM, EMB), F32),
        jax.ShapeDtypeStruct((N_MM, EMB), F32),
        jax.ShapeDtypeStruct((N_DD, EMB), F32),
        jax.ShapeDtypeStruct((N_DD, EMB), F32),
    ],
)


def _tc3_body(*refs):
    hs = []
    for g in range(4):
        s_ref, xw_ref, di2_ref, b_ref = refs[4 * g:4 * g + 4]
        h = s_ref[...] + di2_ref[...] * xw_ref[...] + b_ref[...]
        hs.append(jnp.maximum(h, 0.0))
    wlmf, blmf, wlmg, blmg = refs[16:20]
    mi_ref, dis_ref = refs[20], refs[21]
    mmf = jnp.dot(hs[0], wlmf[...], preferred_element_type=F32) + blmf[...]
    mmg = jnp.dot(hs[1], wlmg[...], preferred_element_type=F32) + blmg[...]
    dds = jnp.dot(hs[2], wlmf[...], preferred_element_type=F32) + blmf[...]
    ddg = jnp.dot(hs[3], wlmg[...], preferred_element_type=F32) + blmg[...]
    mw = (mmf > 0).astype(F32)
    dw = (dds > 0).astype(F32)
    mi_ref[...] = mw * mmf + (1.0 - mw) * mmg
    dis_ref[...] = dw * dds + (1.0 - dw) * ddg


_tc3 = pl.pallas_call(
    _tc3_body,
    out_shape=[
        jax.ShapeDtypeStruct((N_MM, FOUT), F32),
        jax.ShapeDtypeStruct((N_DD, FOUT), F32),
    ],
)


def _tc4_body(mi_ref, dis_ref, o_ref):
    o_ref[...] = lax.dot_general(
        mi_ref[...], dis_ref[...],
        dimension_numbers=(((1,), (1,)), ((), ())),
        preferred_element_type=F32)


_tc4 = pl.pallas_call(
    _tc4_body,
    out_shape=jax.ShapeDtypeStruct((N_MM, N_DD), F32),
)


def kernel(mm0, dd0, mm_func_edges, mm_func_matrix, mm_gip_edges,
           mm_gip_matrix, dd_sema_edges, dd_sema_matrix, dd_gip_edges,
           dd_gip_matrix, W1mf, b1mf, W2mf, b2mf, W1mg, b1mg, W2mg, b2mg,
           W1ds, b1ds, W2ds, b2ds, W1dg, b1dg, W2dg, b2dg,
           Wlmf, blmf, Wlmg, blmg):
    e_mf = mm_func_edges.astype(I32).reshape(2, -1, 128)
    e_mg = mm_gip_edges.astype(I32).reshape(2, -1, 128)
    e_ds = dd_sema_edges.astype(I32).reshape(2, -1, 128)
    e_dg = dd_gip_edges.astype(I32).reshape(2, -1, 128)

    (nrm_mf, di2_mf, nrm_mg, di2_mg, nrm_ds, di2_ds, nrm_dg, di2_dg) = _prep(
        mm_func_matrix.reshape(-1), e_mf,
        mm_gip_matrix.reshape(-1), e_mg,
        dd_sema_matrix.reshape(-1), e_ds,
        dd_gip_matrix.reshape(-1), e_dg)

    xw1_mf, xw1_mg, xw1_ds, xw1_dg = _tc1(mm0, dd0, W1mf, W1mg, W1ds, W1dg)

    sp1_mf, sp1_mg, sp1_ds, sp1_dg = _msg(
        xw1_mf, e_mf, nrm_mf, xw1_mg, e_mg, nrm_mg,
        xw1_ds, e_ds, nrm_ds, xw1_dg, e_dg, nrm_dg)

    di2c_mf = di2_mf.reshape(-1, 1)
    di2c_mg = di2_mg.reshape(-1, 1)
    di2c_ds = di2_ds.reshape(-1, 1)
    di2c_dg = di2_dg.reshape(-1, 1)

    xw2_mf, xw2_mg, xw2_ds, xw2_dg = _tc2(
        sp1_mf, xw1_mf, di2c_mf, b1mf.reshape(1, -1), W2mf,
        sp1_mg, xw1_mg, di2c_mg, b1mg.reshape(1, -1), W2mg,
        sp1_ds, xw1_ds, di2c_ds, b1ds.reshape(1, -1), W2ds,
        sp1_dg, xw1_dg, di2c_dg, b1dg.reshape(1, -1), W2dg)

    sp2_mf, sp2_mg, sp2_ds, sp2_dg = _msg(
        xw2_mf, e_mf, nrm_mf, xw2_mg, e_mg, nrm_mg,
        xw2_ds, e_ds, nrm_ds, xw2_dg, e_dg, nrm_dg)

    mi, dis = _tc3(
        sp2_mf, xw2_mf, di2c_mf, b2mf.reshape(1, -1),
        sp2_mg, xw2_mg, di2c_mg, b2mg.reshape(1, -1),
        sp2_ds, xw2_ds, di2c_ds, b2ds.reshape(1, -1),
        sp2_dg, xw2_dg, di2c_dg, b2dg.reshape(1, -1),
        Wlmf, blmf.reshape(1, -1), Wlmg, blmg.reshape(1, -1))

    prod = _tc4(mi, dis)
    return (prod, mi, dis)


# speedup vs baseline: 1.0130x; 1.0031x over previous
import functools

import jax
import jax.numpy as jnp
from jax import lax
from jax.experimental import pallas as pl
from jax.experimental.pallas import tpu as pltpu
from jax.experimental.pallas import tpu_sc as plsc

EMB = 128
FOUT = 64
N_MM = 4096
N_DD = 2048
E_MM = 131072
E_DD = 65536

NC = 2
NS = 16

F32 = jnp.float32
I32 = jnp.int32

_mesh = plsc.VectorSubcoreMesh(
    core_axis_name="c", subcore_axis_name="s", num_cores=NC, num_subcores=NS)

_ZERO16F = functools.partial(jnp.zeros, (16,), F32)


def _newton_rsqrt(x):
    i = lax.bitcast_convert_type(x, I32)
    i = jnp.int32(0x5F3759DF) - (i >> 1)
    y = lax.bitcast_convert_type(i, F32)
    for _ in range(3):
        y = y * (1.5 - 0.5 * x * y * y)
    return y


def _prep_body(mat_mf, e_mf, mat_mg, e_mg, mat_ds, e_ds, mat_dg, e_dg,
               nrm_mf_o, di2_mf_o, nrm_mg_o, di2_mg_o,
               nrm_ds_o, di2_ds_o, nrm_dg_o, di2_dg_o,
               ix_v, dd_v, ew_v, zz_v, pa_v, dv_v, deg4k, deg2k, sem):
    cid = lax.axis_index("c")
    sid = lax.axis_index("s")

    def _zz(i, carry):
        zz_v[pl.ds(i * 16, 16)] = _ZERO16F()
        return carry
    lax.fori_loop(0, 16, _zz, 0)

    for degsh, n in ((deg4k, N_MM), (deg2k, N_DD)):
        chunk = n // NS
        pltpu.sync_copy(zz_v.at[pl.ds(0, chunk)],
                        degsh.at[pl.ds(sid * chunk, chunk)])
    plsc.subcore_barrier()

    def _graph(mat, e_hbm, nrm_o, di2_o, degsh, n, e):
        shift = n.bit_length() - 1
        nrows = e // 128 // NS
        rbase = sid * nrows
        pltpu.sync_copy(e_hbm.at[0, pl.ds(rbase, nrows)],
                        ix_v.at[pl.ds(0, nrows)])
        pltpu.sync_copy(e_hbm.at[1, pl.ds(rbase, nrows)],
                        dd_v.at[pl.ds(0, nrows)])

        def _flat(r):
            for k in range(8):
                sl = pl.ds(k * 16, 16)
                ix_v[r, sl] = ix_v[r, sl] * n + dd_v[r, sl]
        plsc.parallel_loop(0, nrows, 1, unroll=4)(_flat)

        def _fire(j, carry):
            pltpu.async_copy(mat.at[ix_v.at[j]], ew_v.at[j], sem)
            return carry
        lax.fori_loop(0, nrows, _fire, 0)

        def _drain(j, carry):
            pltpu.make_async_copy(mat.at[ix_v.at[j]], ew_v.at[j], sem).wait()
            return carry
        lax.fori_loop(0, nrows, _drain, 0)

        def _scat(j, carry):
            pltpu.sync_copy(ew_v.at[j], degsh.at[dd_v.at[j]], add=True)
            return carry
        lax.fori_loop(0, nrows, _scat, 0)
        plsc.subcore_barrier()

        pltpu.sync_copy(degsh, pa_v.at[pl.ds(0, n)])

        def _dv(i):
            sl = pl.ds(i * 16, 16)
            x = pa_v[sl] + 1.0
            y = _newton_rsqrt(x)
            dv_v[sl] = y
            pa_v[sl] = y * y
        plsc.parallel_loop(0, n // 16, 1, unroll=4)(_dv)

        @pl.when(sid == 0)
        def _():
            pltpu.sync_copy(pa_v.at[pl.ds(0, n)], di2_o)

        def _nrow(r):
            for k in range(8):
                sl = pl.ds(k * 16, 16)
                s16 = lax.shift_right_logical(ix_v[r, sl], shift)
                dsv = plsc.load_gather(dv_v, [s16])
                ddv = plsc.load_gather(dv_v, [dd_v[r, sl]])
                ew_v[r, sl] = dsv * ew_v[r, sl] * ddv
        plsc.parallel_loop(0, nrows, 1, unroll=4)(_nrow)

        pltpu.sync_copy(ew_v.at[pl.ds(0, nrows)], nrm_o.at[pl.ds(rbase, nrows)])

    @pl.when(cid == 0)
    def _c0():
        _graph(mat_mf, e_mf, nrm_mf_o, di2_mf_o, deg4k, N_MM, E_MM)
        _graph(mat_ds, e_ds, nrm_ds_o, di2_ds_o, deg2k, N_DD, E_DD)

    @pl.when(cid == 1)
    def _c1():
        _graph(mat_mg, e_mg, nrm_mg_o, di2_mg_o, deg4k, N_MM, E_MM)
        _graph(mat_dg, e_dg, nrm_dg_o, di2_dg_o, deg2k, N_DD, E_DD)


_prep = pl.kernel(
    _prep_body,
    out_type=[
        jax.ShapeDtypeStruct((E_MM // 128, 128), F32),
        jax.ShapeDtypeStruct((N_MM,), F32),
        jax.ShapeDtypeStruct((E_MM // 128, 128), F32),
        jax.ShapeDtypeStruct((N_MM,), F32),
        jax.ShapeDtypeStruct((E_DD // 128, 128), F32),
        jax.ShapeDtypeStruct((N_DD,), F32),
        jax.ShapeDtypeStruct((E_DD // 128, 128), F32),
        jax.ShapeDtypeStruct((N_DD,), F32),
    ],
    mesh=_mesh,
    compiler_params=pltpu.CompilerParams(needs_layout_passes=False),
    scratch_types=[
        pltpu.VMEM((E_MM // 128 // NS, 128), I32),
        pltpu.VMEM((E_MM // 128 // NS, 128), I32),
        pltpu.VMEM((E_MM // 128 // NS, 128), F32),
        pltpu.VMEM((256,), F32),
        pltpu.VMEM((N_MM,), F32),
        pltpu.VMEM((N_MM,), F32),
        pltpu.VMEM_SHARED((N_MM,), F32),
        pltpu.VMEM_SHARED((N_DD,), F32),
        pltpu.SemaphoreType.DMA,
    ],
)


def _msg_body(xw_mf, e_mf, n_mf, xw_mg, e_mg, n_mg,
              xw_ds, e_ds, n_ds, xw_dg, e_dg, n_dg,
              o_mf, o_mg, o_ds, o_dg,
              si_v, di_v, nw_v, rows_a, rows_b, sc_a, sc_b, z_v,
              acc4k,
              gsem_a, gsem_b, ssem_a, ssem_b):
    cid = lax.axis_index("c")
    sid = lax.axis_index("s")

    def _zrow(r, carry):
        for k in range(8):
            z_v[r, pl.ds(k * 16, 16)] = _ZERO16F()
        return carry
    lax.fori_loop(0, 32, _zrow, 0)

    def _zero_own(acc, n):
        rpt = n // NS

        def _zb(b, carry):
            pltpu.sync_copy(z_v, acc.at[pl.ds(sid * rpt + b * 32, 32)])
            return carry
        lax.fori_loop(0, rpt // 32, _zb, 0)

    _zero_own(acc4k, N_MM)
    plsc.subcore_barrier()

    bufs = ((rows_a, sc_a, gsem_a, ssem_a), (rows_b, sc_b, gsem_b, ssem_b))

    def _graph(xw_hbm, e_hbm, n_hbm, acc, e):
        nrows = e // 128 // NS
        rbase = sid * nrows
        pltpu.sync_copy(e_hbm.at[0, pl.ds(rbase, nrows)],
                        si_v.at[pl.ds(0, nrows)])
        pltpu.sync_copy(e_hbm.at[1, pl.ds(rbase, nrows)],
                        di_v.at[pl.ds(0, nrows)])
        pltpu.sync_copy(n_hbm.at[pl.ds(rbase, nrows)],
                        nw_v.at[pl.ds(0, nrows)])

        pltpu.async_copy(xw_hbm.at[si_v.at[0]], rows_a, gsem_a)
        pltpu.async_copy(xw_hbm.at[si_v.at[1]], rows_b, gsem_b)

        def _pair(j2, carry):
            for b, (rows, scb, gsem, ssem) in enumerate(bufs):
                j = j2 * 2 + b
                pltpu.make_async_copy(xw_hbm.at[si_v.at[j]], rows, gsem).wait()

                @pl.when(j2 > 0)
                def _ws(scb=scb, ssem=ssem, j=j):
                    pltpu.make_async_copy(
                        scb, acc.at[di_v.at[j - 2]], ssem).wait()

                def _rs(r, rows=rows, scb=scb, j=j):
                    wv = plsc.load_gather(
                        nw_v, [jnp.full((16,), j, I32),
                               jnp.full((16,), r, I32)])
                    for k in range(8):
                        sl = pl.ds(k * 16, 16)
                        scb[r, sl] = rows[r, sl] * wv
                plsc.parallel_loop(0, 128, 1, unroll=16)(_rs)

                @pl.when(j + 2 < nrows)
                def _g2(j=j, rows=rows, gsem=gsem):
                    pltpu.async_copy(xw_hbm.at[si_v.at[j + 2]], rows, gsem)

                pltpu.async_copy(scb, acc.at[di_v.at[j]], ssem, add=True)
            return carry
        lax.fori_loop(0, nrows // 2, _pair, 0)

        pltpu.make_async_copy(sc_a, acc.at[di_v.at[nrows - 2]], ssem_a).wait()
        pltpu.make_async_copy(sc_b, acc.at[di_v.at[nrows - 1]], ssem_b).wait()

    def _writeout_own(acc, out, n):
        rpt = n // NS
        pltpu.sync_copy(acc.at[pl.ds(sid * rpt, rpt)],
                        out.at[pl.ds(sid * rpt, rpt)])

    @pl.when(cid == 0)
    def _g0():
        _graph(xw_mf, e_mf, n_mf, acc4k, E_MM)

    @pl.when(cid == 1)
    def _g1():
        _graph(xw_mg, e_mg, n_mg, acc4k, E_MM)
    plsc.subcore_barrier()

    @pl.when(cid == 0)
    def _w0():
        _writeout_own(acc4k, o_mf, N_MM)

    @pl.when(cid == 1)
    def _w1():
        _writeout_own(acc4k, o_mg, N_MM)
    plsc.subcore_barrier()
    _zero_own(acc4k, N_DD)
    plsc.subcore_barrier()

    @pl.when(cid == 0)
    def _g0d():
        _graph(xw_ds, e_ds, n_ds, acc4k, E_DD)

    @pl.when(cid == 1)
    def _g1d():
        _graph(xw_dg, e_dg, n_dg, acc4k, E_DD)
    plsc.subcore_barrier()

    @pl.when(cid == 0)
    def _w0d():
        _writeout_own(acc4k, o_ds, N_DD)

    @pl.when(cid == 1)
    def _w1d():
        _writeout_own(acc4k, o_dg, N_DD)


_msg = pl.kernel(
    _msg_body,
    out_type=[
        jax.ShapeDtypeStruct((N_MM, EMB), F32),
        jax.ShapeDtypeStruct((N_MM, EMB), F32),
        jax.ShapeDtypeStruct((N_DD, EMB), F32),
        jax.ShapeDtypeStruct((N_DD, EMB), F32),
    ],
    mesh=_mesh,
    compiler_params=pltpu.CompilerParams(needs_layout_passes=False),
    scratch_types=[
        pltpu.VMEM((E_MM // 128 // NS, 128), I32),
        pltpu.VMEM((E_MM // 128 // NS, 128), I32),
        pltpu.VMEM((E_MM // 128 // NS, 128), F32),
        pltpu.VMEM((128, EMB), F32),
        pltpu.VMEM((128, EMB), F32),
        pltpu.VMEM((128, EMB), F32),
        pltpu.VMEM((128, EMB), F32),
        pltpu.VMEM((32, EMB), F32),
        pltpu.VMEM_SHARED((N_MM, EMB), F32),
        pltpu.SemaphoreType.DMA,
        pltpu.SemaphoreType.DMA,
        pltpu.SemaphoreType.DMA,
        pltpu.SemaphoreType.DMA,
    ],
)


def _tc1_body(a_ref, b_ref, wmf, wmg, wds, wdg, omf, omg, ods, odg):
    a = a_ref[...]
    b = b_ref[...]
    omf[...] = jnp.dot(a, wmf[...], preferred_element_type=F32)
    omg[...] = jnp.dot(a, wmg[...], preferred_element_type=F32)
    ods[...] = jnp.dot(b, wds[...], preferred_element_type=F32)
    odg[...] = jnp.dot(b, wdg[...], preferred_element_type=F32)


_tc1 = pl.pallas_call(
    _tc1_body,
    out_shape=[
        jax.ShapeDtypeStruct((N_MM, EMB), F32),
        jax.ShapeDtypeStruct((N_MM, EMB), F32),
        jax.ShapeDtypeStruct((N_DD, EMB), F32),
        jax.ShapeDtypeStruct((N_DD, EMB), F32),
    ],
)


def _tc2_body(*refs):
    for g in range(4):
        s_ref, xw_ref, di2_ref, b_ref, w_ref = refs[5 * g:5 * g + 5]
        o_ref = refs[20 + g]
        h = s_ref[...] + di2_ref[...] * xw_ref[...] + b_ref[...]
        h = jnp.maximum(h, 0.0)
        o_ref[...] = jnp.dot(h, w_ref[...], preferred_element_type=F32)


_tc2 = pl.pallas_call(
    _tc2_body,
    out_shape=[
        jax.ShapeDtypeStruct((N_MM, EMB), F32),
        jax.ShapeDtypeStruct((N_MM, EMB), F32),
        jax.ShapeDtypeStruct((N_DD, EMB), F32),
        jax.ShapeDtypeStruct((N_DD, EMB), F32),
    ],
)


def _tc3_body(*refs):
    hs = []
    for g in range(4):
        s_ref, xw_ref, di2_ref, b_ref = refs[4 * g:4 * g + 4]
        h = s_ref[...] + di2_ref[...] * xw_ref[...] + b_ref[...]
        hs.append(jnp.maximum(h, 0.0))
    wlmf, blmf, wlmg, blmg = refs[16:20]
    mi_ref, dis_ref = refs[20], refs[21]
    mmf = jnp.dot(hs[0], wlmf[...], preferred_element_type=F32) + blmf[...]
    mmg = jnp.dot(hs[1], wlmg[...], preferred_element_type=F32) + blmg[...]
    dds = jnp.dot(hs[2], wlmf[...], preferred_element_type=F32) + blmf[...]
    ddg = jnp.dot(hs[3], wlmg[...], preferred_element_type=F32) + blmg[...]
    mw = (mmf > 0).astype(F32)
    dw = (dds > 0).astype(F32)
    mi_ref[...] = mw * mmf + (1.0 - mw) * mmg
    dis_ref[...] = dw * dds + (1.0 - dw) * ddg


_tc3 = pl.pallas_call(
    _tc3_body,
    out_shape=[
        jax.ShapeDtypeStruct((N_MM, FOUT), F32),
        jax.ShapeDtypeStruct((N_DD, FOUT), F32),
    ],
)


def _tc4_body(mi_ref, dis_ref, o_ref):
    o_ref[...] = lax.dot_general(
        mi_ref[...], dis_ref[...],
        dimension_numbers=(((1,), (1,)), ((), ())),
        preferred_element_type=F32)


_tc4 = pl.pallas_call(
    _tc4_body,
    out_shape=jax.ShapeDtypeStruct((N_MM, N_DD), F32),
)


def kernel(mm0, dd0, mm_func_edges, mm_func_matrix, mm_gip_edges,
           mm_gip_matrix, dd_sema_edges, dd_sema_matrix, dd_gip_edges,
           dd_gip_matrix, W1mf, b1mf, W2mf, b2mf, W1mg, b1mg, W2mg, b2mg,
           W1ds, b1ds, W2ds, b2ds, W1dg, b1dg, W2dg, b2dg,
           Wlmf, blmf, Wlmg, blmg):
    e_mf = mm_func_edges.astype(I32).reshape(2, -1, 128)
    e_mg = mm_gip_edges.astype(I32).reshape(2, -1, 128)
    e_ds = dd_sema_edges.astype(I32).reshape(2, -1, 128)
    e_dg = dd_gip_edges.astype(I32).reshape(2, -1, 128)

    (nrm_mf, di2_mf, nrm_mg, di2_mg, nrm_ds, di2_ds, nrm_dg, di2_dg) = _prep(
        mm_func_matrix.reshape(-1), e_mf,
        mm_gip_matrix.reshape(-1), e_mg,
        dd_sema_matrix.reshape(-1), e_ds,
        dd_gip_matrix.reshape(-1), e_dg)

    xw1_mf, xw1_mg, xw1_ds, xw1_dg = _tc1(mm0, dd0, W1mf, W1mg, W1ds, W1dg)

    sp1_mf, sp1_mg, sp1_ds, sp1_dg = _msg(
        xw1_mf, e_mf, nrm_mf, xw1_mg, e_mg, nrm_mg,
        xw1_ds, e_ds, nrm_ds, xw1_dg, e_dg, nrm_dg)

    di2c_mf = di2_mf.reshape(-1, 1)
    di2c_mg = di2_mg.reshape(-1, 1)
    di2c_ds = di2_ds.reshape(-1, 1)
    di2c_dg = di2_dg.reshape(-1, 1)

    xw2_mf, xw2_mg, xw2_ds, xw2_dg = _tc2(
        sp1_mf, xw1_mf, di2c_mf, b1mf.reshape(1, -1), W2mf,
        sp1_mg, xw1_mg, di2c_mg, b1mg.reshape(1, -1), W2mg,
        sp1_ds, xw1_ds, di2c_ds, b1ds.reshape(1, -1), W2ds,
        sp1_dg, xw1_dg, di2c_dg, b1dg.reshape(1, -1), W2dg)

    sp2_mf, sp2_mg, sp2_ds, sp2_dg = _msg(
        xw2_mf, e_mf, nrm_mf, xw2_mg, e_mg, nrm_mg,
        xw2_ds, e_ds, nrm_ds, xw2_dg, e_dg, nrm_dg)

    mi, dis = _tc3(
        sp2_mf, xw2_mf, di2c_mf, b2mf.reshape(1, -1),
        sp2_mg, xw2_mg, di2c_mg, b2mg.reshape(1, -1),
        sp2_ds, xw2_ds, di2c_ds, b2ds.reshape(1, -1),
        sp2_dg, xw2_dg, di2c_dg, b2dg.reshape(1, -1),
        Wlmf, blmf.reshape(1, -1), Wlmg, blmg.reshape(1, -1))

    prod = _tc4(mi, dis)
    return (prod, mi, dis)
